# baseline (speedup 1.0000x reference)
import functools

import jax
import jax.numpy as jnp
from jax import lax
from jax.experimental import pallas as pl
from jax.experimental.pallas import tpu as pltpu
from jax.experimental.pallas import tpu_sc as plsc

D = 6
HIDDEN = 64
N_TAB = 10000
NNZ = 320000
OUTW = 30
TABW = 32

NW = 32
PER_W = NNZ // NW
BLK = 400
NBLK = PER_W // BLK
CHUNK = 80
NCHUNK = BLK // CHUNK
NGRP = BLK // 16
MAPW = 36

_TBLK = 2000


def _tc_tables_body(x_ref, e_ref, wpx_ref, wqx_ref, wpe_ref, wqe_ref,
                    px_ref, pe_ref):
    hi = lax.Precision.HIGHEST
    mx = jnp.mean(x_ref[...], axis=1)
    px_ref[...] = (jnp.dot(mx, wpx_ref[...], precision=hi)
                   + jnp.dot(mx * mx, wqx_ref[...], precision=hi))
    me = jnp.mean(e_ref[...], axis=1)
    pe_ref[...] = (jnp.dot(me, wpe_ref[...], precision=hi)
                   + jnp.dot(me * me, wqe_ref[...], precision=hi))


def _build_tables(xr, er, wpx, wqx, wpe, wqe):
    blk = pl.BlockSpec((_TBLK, D, HIDDEN), lambda i: (i, 0, 0))
    wspec = pl.BlockSpec((HIDDEN, TABW), lambda i: (0, 0))
    ospec = pl.BlockSpec((_TBLK, TABW), lambda i: (i, 0))
    return pl.pallas_call(
        _tc_tables_body,
        grid=(N_TAB // _TBLK,),
        in_specs=[blk, blk, wspec, wspec, wspec, wspec],
        out_specs=[ospec, ospec],
        out_shape=[jax.ShapeDtypeStruct((N_TAB, TABW), jnp.float32),
                   jax.ShapeDtypeStruct((N_TAB, TABW), jnp.float32)],
    )(xr, er, wpx, wqx, wpe, wqe)


def _rsqrt16(v):
    i = plsc.bitcast(v, jnp.int32)
    i = jnp.int32(0x5F3759DF) - (i >> 1)
    y = plsc.bitcast(i, jnp.float32)
    for _ in range(3):
        y = y * (1.5 - 0.5 * v * y * y)
    return y


def _sc_body(px_hbm, pe_hbm, row_hbm, col_hbm, nbc_hbm, out_hbm,
             rowv, colv, gp, ob, bcv, gsem, asem, osem, csem):
    wid = lax.axis_index("s") * 2 + lax.axis_index("c")
    pltpu.async_copy(nbc_hbm, bcv, csem).wait()
    lane = lax.iota(jnp.int32, 16)

    def start_fn(b):
        return pl.multiple_of(wid * PER_W + b * BLK, 8)

    def copy_idx(b, s):
        start = start_fn(b)
        pltpu.sync_copy(row_hbm.at[pl.ds(start, BLK)], rowv.at[s])
        pltpu.sync_copy(col_hbm.at[pl.ds(start, BLK)], colv.at[s])

    def px_copies(s):
        return [pltpu.make_async_copy(
                    px_hbm.at[rowv.at[s, pl.ds(i * CHUNK, CHUNK)]],
                    gp.at[s, pl.ds(i * CHUNK, CHUNK)], gsem)
                for i in range(NCHUNK)]

    def pe_copies(s):
        return [pltpu.make_async_copy(
                    pe_hbm.at[colv.at[s, pl.ds(i * CHUNK, CHUNK)]],
                    gp.at[s, pl.ds(i * CHUNK, CHUNK)], asem)
                for i in range(NCHUNK)]

    def fire_px(s):
        for cp in px_copies(s):
            cp.start()

    def drain_px_fire_pe(s):
        for cp in px_copies(s):
            cp.wait()
        for i in range(NCHUNK):
            pltpu.async_copy(pe_hbm.at[colv.at[s, pl.ds(i * CHUNK, CHUNK)]],
                             gp.at[s, pl.ds(i * CHUNK, CHUNK)], asem, add=True)

    def drain_pe(s):
        for cp in pe_copies(s):
            cp.wait()

    def compute_groups(g_lo, g_hi, s):

        def group_body(g, gc):
            base = lane + g * 16
            ps = [plsc.load_gather(gp, [jnp.full((16,), s, jnp.int32), base,
                                        jnp.full((16,), j, jnp.int32)])
                  for j in range(TABW)]
            mu = ps[30]
            msq = ps[31]
            inv = _rsqrt16(msq - mu * mu + 1e-5)
            ninv = -inv
            h = []
            for j in range(OUTW):
                t = jnp.exp(ninv * ps[j] + bcv[j])
                h.append(1.0 / (1.0 + t))
            base36 = base * MAPW
            for d in range(6):
                for k in range(6):
                    m = h[2 * d] * h[12 + k] + h[2 * d + 1] * h[18 + k]
                    if d == k:
                        m = m + h[24 + d]
                    plsc.store_scatter(
                        ob, [jnp.full((16,), s, jnp.int32),
                             base36 + jnp.full((16,), d * 6 + k, jnp.int32)], m)
            return gc

        lax.fori_loop(g_lo, g_hi, group_body, 0)

    copy_idx(0, 0)
    fire_px(0)
    drain_px_fire_pe(0)
    copy_idx(1, 1)
    fire_px(1)

    def block_body(b, carry):
        s = lax.rem(b, 2)
        start = start_fn(b)
        drain_pe(s)

        @pl.when(b >= 2)
        def _():
            pltpu.make_async_copy(
                ob.at[s],
                out_hbm.at[pl.ds(start_fn(b - 2) * MAPW, BLK * MAPW)],
                osem).wait()

        compute_groups(0, NGRP // 2, s)

        @pl.when(b + 1 < NBLK)
        def _():
            drain_px_fire_pe(1 - s)

        compute_groups(NGRP // 2, NGRP, s)
        pltpu.async_copy(ob.at[s],
                         out_hbm.at[pl.ds(start * MAPW, BLK * MAPW)], osem)

        @pl.when(b + 2 < NBLK)
        def _():
            copy_idx(b + 2, s)
            fire_px(s)

        return carry

    lax.fori_loop(0, NBLK, block_body, 0)
    for b in (NBLK - 2, NBLK - 1):
        s = b % 2
        pltpu.make_async_copy(
            ob.at[s], out_hbm.at[pl.ds(start_fn(b) * MAPW, BLK * MAPW)],
            osem).wait()


def _sc_maps(px_tab, pe_tab, row, col, nbc):
    mesh = plsc.VectorSubcoreMesh(core_axis_name="c", subcore_axis_name="s",
                                  num_cores=2, num_subcores=16)
    fn = functools.partial(
        pl.kernel,
        out_type=jax.ShapeDtypeStruct((NNZ * MAPW,), jnp.float32),
        mesh=mesh,
        compiler_params=pltpu.CompilerParams(needs_layout_passes=False,
                                             use_tc_tiling_on_sc=False),
        scratch_types=[
            pltpu.VMEM((2, BLK), jnp.int32),
            pltpu.VMEM((2, BLK), jnp.int32),
            pltpu.VMEM((2, BLK, TABW), jnp.float32),
            pltpu.VMEM((2, BLK * MAPW), jnp.float32),
            pltpu.VMEM((OUTW, 16), jnp.float32),
            pltpu.SemaphoreType.DMA,
            pltpu.SemaphoreType.DMA,
            pltpu.SemaphoreType.DMA,
            pltpu.SemaphoreType.DMA,
        ],
    )(_sc_body)
    return fn(px_tab, pe_tab, row, col, nbc)


def kernel(x, e, hyperedge_index, node_types, hyperedge_types,
           ln_gamma, ln_beta, W, b):
    H = HIDDEN
    scale = jnp.float32(1.0 / (2 * H))
    G = ln_gamma @ W
    bc30 = ln_beta @ W + b
    nbc = jnp.tile((-bc30)[:, None], (1, 16)).astype(jnp.float32)

    def make_wp(gam_half, w_half):
        wp = gam_half[:, None] * w_half - G[None, :] * scale
        wp = jnp.concatenate(
            [wp, jnp.full((H, 1), scale), jnp.zeros((H, 1))], axis=1)
        wq = jnp.concatenate(
            [jnp.zeros((H, OUTW + 1)), jnp.full((H, 1), scale)], axis=1)
        return wp.astype(jnp.float32), wq.astype(jnp.float32)

    wpx, wqx = make_wp(ln_gamma[:H], W[:H])
    wpe, wqe = make_wp(ln_gamma[H:], W[H:])
    px_tab, pe_tab = _build_tables(x.reshape(N_TAB, D, H),
                                   e.reshape(N_TAB, D, H),
                                   wpx, wqx, wpe, wqe)

    row = hyperedge_index[0].reshape(NNZ)
    col = hyperedge_index[1].reshape(NNZ)
    maps = _sc_maps(px_tab, pe_tab, row, col, nbc)
    return maps.reshape(NNZ, D, D)


# speedup vs baseline: 2.1570x; 2.1570x over previous
import functools

import jax
import jax.numpy as jnp
from jax import lax
from jax.experimental import pallas as pl
from jax.experimental.pallas import tpu as pltpu
from jax.experimental.pallas import tpu_sc as plsc

D = 6
HIDDEN = 64
N_TAB = 10000
NNZ = 320000
OUTW = 30
TABW = 32

NW = 32
PER_W = NNZ // NW
BLK = 400
NBLK = PER_W // BLK
CHUNK = 80
NCHUNK = BLK // CHUNK
NGRP = BLK // 16
MAPW = 36

_TBLK = 2000


def _tc_tables_body(x_ref, e_ref, wpx_ref, wqx_ref, wpe_ref, wqe_ref,
                    px_ref, pe_ref):
    hi = lax.Precision.HIGHEST
    mx = jnp.mean(x_ref[...], axis=1)
    px_ref[...] = (jnp.dot(mx, wpx_ref[...], precision=hi)
                   + jnp.dot(mx * mx, wqx_ref[...], precision=hi))
    me = jnp.mean(e_ref[...], axis=1)
    pe_ref[...] = (jnp.dot(me, wpe_ref[...], precision=hi)
                   + jnp.dot(me * me, wqe_ref[...], precision=hi))


def _build_tables(xr, er, wpx, wqx, wpe, wqe):
    blk = pl.BlockSpec((_TBLK, D, HIDDEN), lambda i: (i, 0, 0))
    wspec = pl.BlockSpec((HIDDEN, TABW), lambda i: (0, 0))
    ospec = pl.BlockSpec((_TBLK, TABW), lambda i: (i, 0))
    return pl.pallas_call(
        _tc_tables_body,
        grid=(N_TAB // _TBLK,),
        in_specs=[blk, blk, wspec, wspec, wspec, wspec],
        out_specs=[ospec, ospec],
        out_shape=[jax.ShapeDtypeStruct((N_TAB, TABW), jnp.float32),
                   jax.ShapeDtypeStruct((N_TAB, TABW), jnp.float32)],
    )(xr, er, wpx, wqx, wpe, wqe)


def _rsqrt16(v):
    i = plsc.bitcast(v, jnp.int32)
    i = jnp.int32(0x5F3759DF) - (i >> 1)
    y = plsc.bitcast(i, jnp.float32)
    for _ in range(3):
        y = y * (1.5 - 0.5 * v * y * y)
    return y


def _sc_body(px_hbm, pe_hbm, row_hbm, col_hbm, nbc_hbm, out_hbm,
             rowv, colv, gp, ob, bcv, gsem, asem, osem, csem):
    wid = lax.axis_index("s") * 2 + lax.axis_index("c")
    pltpu.async_copy(nbc_hbm, bcv, csem).wait()
    lane = lax.iota(jnp.int32, 16)

    def start_fn(b):
        return pl.multiple_of(wid * PER_W + b * BLK, 8)

    def copy_idx(b, s):
        start = start_fn(b)
        pltpu.sync_copy(row_hbm.at[pl.ds(start, BLK)], rowv.at[s])
        pltpu.sync_copy(col_hbm.at[pl.ds(start, BLK)], colv.at[s])

    def px_copies(s):
        return [pltpu.make_async_copy(
                    px_hbm.at[rowv.at[s, pl.ds(i * CHUNK, CHUNK)]],
                    gp.at[s, pl.ds(i * CHUNK, CHUNK)], gsem)
                for i in range(NCHUNK)]

    def pe_copies(s):
        return [pltpu.make_async_copy(
                    pe_hbm.at[colv.at[s, pl.ds(i * CHUNK, CHUNK)]],
                    gp.at[s, pl.ds(i * CHUNK, CHUNK)], asem)
                for i in range(NCHUNK)]

    def fire_px(s):
        for cp in px_copies(s):
            cp.start()

    def drain_px_fire_pe(s):
        for cp in px_copies(s):
            cp.wait()
        for i in range(NCHUNK):
            pltpu.async_copy(pe_hbm.at[colv.at[s, pl.ds(i * CHUNK, CHUNK)]],
                             gp.at[s, pl.ds(i * CHUNK, CHUNK)], asem, add=True)

    def drain_pe(s):
        for cp in pe_copies(s):
            cp.wait()

    def compute_groups(g_lo, g_hi, s):

        @plsc.parallel_loop(g_lo, g_hi, unroll=2)
        def group_body(g):
            base = lane + g * 16
            ps = [plsc.load_gather(gp, [jnp.full((16,), s, jnp.int32), base,
                                        jnp.full((16,), j, jnp.int32)])
                  for j in range(TABW)]
            mu = ps[30]
            msq = ps[31]
            inv = _rsqrt16(msq - mu * mu + 1e-5)
            ninv = -inv
            h = []
            for j in range(OUTW):
                t = jnp.exp(ninv * ps[j] + bcv[j])
                h.append(1.0 / (1.0 + t))
            for d in range(6):
                for k in range(6):
                    m = h[2 * d] * h[12 + k] + h[2 * d + 1] * h[18 + k]
                    if d == k:
                        m = m + h[24 + d]
                    plsc.store_scatter(
                        ob, [jnp.full((16,), s, jnp.int32), base,
                             jnp.full((16,), d * 6 + k, jnp.int32)], m)

    copy_idx(0, 0)
    fire_px(0)
    drain_px_fire_pe(0)
    copy_idx(1, 1)
    fire_px(1)

    def block_body(b, carry):
        s = lax.rem(b, 2)
        start = start_fn(b)
        drain_pe(s)

        @pl.when(b >= 2)
        def _():
            pltpu.make_async_copy(
                ob.at[s], out_hbm.at[pl.ds(start_fn(b - 2), BLK)], osem).wait()

        compute_groups(0, NGRP // 2, s)

        @pl.when(b + 1 < NBLK)
        def _():
            drain_px_fire_pe(1 - s)

        compute_groups(NGRP // 2, NGRP, s)
        pltpu.async_copy(ob.at[s], out_hbm.at[pl.ds(start, BLK)], osem)

        @pl.when(b + 2 < NBLK)
        def _():
            copy_idx(b + 2, s)
            fire_px(s)

        return carry

    lax.fori_loop(0, NBLK, block_body, 0)
    for b in (NBLK - 2, NBLK - 1):
        s = b % 2
        pltpu.make_async_copy(
            ob.at[s], out_hbm.at[pl.ds(start_fn(b), BLK)], osem).wait()


def _sc_maps(px_tab, pe_tab, row, col, nbc):
    mesh = plsc.VectorSubcoreMesh(core_axis_name="c", subcore_axis_name="s",
                                  num_cores=2, num_subcores=16)
    fn = functools.partial(
        pl.kernel,
        out_type=jax.ShapeDtypeStruct((NNZ, MAPW), jnp.float32),
        mesh=mesh,
        compiler_params=pltpu.CompilerParams(needs_layout_passes=False,
                                             use_tc_tiling_on_sc=False),
        scratch_types=[
            pltpu.VMEM((2, BLK), jnp.int32),
            pltpu.VMEM((2, BLK), jnp.int32),
            pltpu.VMEM((2, BLK, TABW), jnp.float32),
            pltpu.VMEM((2, BLK, MAPW), jnp.float32),
            pltpu.VMEM((OUTW, 16), jnp.float32),
            pltpu.SemaphoreType.DMA,
            pltpu.SemaphoreType.DMA,
            pltpu.SemaphoreType.DMA,
            pltpu.SemaphoreType.DMA,
        ],
    )(_sc_body)
    return fn(px_tab, pe_tab, row, col, nbc)


def kernel(x, e, hyperedge_index, node_types, hyperedge_types,
           ln_gamma, ln_beta, W, b):
    H = HIDDEN
    scale = jnp.float32(1.0 / (2 * H))
    G = ln_gamma @ W
    bc30 = ln_beta @ W + b
    nbc = jnp.tile((-bc30)[:, None], (1, 16)).astype(jnp.float32)

    def make_wp(gam_half, w_half):
        wp = gam_half[:, None] * w_half - G[None, :] * scale
        wp = jnp.concatenate(
            [wp, jnp.full((H, 1), scale), jnp.zeros((H, 1))], axis=1)
        wq = jnp.concatenate(
            [jnp.zeros((H, OUTW + 1)), jnp.full((H, 1), scale)], axis=1)
        return wp.astype(jnp.float32), wq.astype(jnp.float32)

    wpx, wqx = make_wp(ln_gamma[:H], W[:H])
    wpe, wqe = make_wp(ln_gamma[H:], W[H:])
    px_tab, pe_tab = _build_tables(x.reshape(N_TAB, D, H),
                                   e.reshape(N_TAB, D, H),
                                   wpx, wqx, wpe, wqe)

    row = hyperedge_index[0].reshape(NNZ)
    col = hyperedge_index[1].reshape(NNZ)
    maps = _sc_maps(px_tab, pe_tab, row, col, nbc)
    return maps.reshape(NNZ, D, D)
